# baseline (speedup 1.0000x reference)
import jax
import jax.numpy as jnp
from jax.experimental import pallas as pl
from jax.experimental.pallas import tpu as pltpu

_R = 20000
_ROWS = 160
_LANES = 128
_RP = _ROWS * _LANES
_IOU_T = 0.7
_HALF_BATCH = 200
_SIGMA = 10.0


def _ohem_kernel(cls0_ref, cls1_ref, ct_ref, lp0_ref, lp1_ref, lt0_ref,
                 lt1_ref, ax1_ref, ay1_ref, ax2_ref, ay2_ref,
                 cls_out, loc_out,
                 ce_ref, sl_ref, areas_ref, keyp_ref, keyn_ref):
    shape = (_ROWS, _LANES)
    lin = (jax.lax.broadcasted_iota(jnp.int32, shape, 0) * _LANES
           + jax.lax.broadcasted_iota(jnp.int32, shape, 1))
    lane_iota = jax.lax.broadcasted_iota(jnp.int32, (1, _LANES), 1)
    zero = jnp.float32(0.0)
    t = ct_ref[...]

    c0 = cls0_ref[...]
    c1 = cls1_ref[...]
    mx = jnp.maximum(c0, c1)
    s0 = c0 - mx
    s1 = c1 - mx
    lse = jnp.log(jnp.exp(s0) + jnp.exp(s1))
    ce = lse - jnp.where(t == 1, s1, s0)

    def _sl1(d):
        less_one = (d < 1.0 / _SIGMA).astype(jnp.float32)
        return (less_one * 0.5 * d ** 2 * _SIGMA
                + jnp.abs(1 - less_one) * (d - 0.5 / _SIGMA))

    sl = (_sl1(jnp.abs(lt0_ref[...] - lp0_ref[...]))
          + _sl1(jnp.abs(lt1_ref[...] - lp1_ref[...])))

    areas_ref[...] = ((ax2_ref[...] - ax1_ref[...])
                      * (ay2_ref[...] - ay1_ref[...]))
    ce_ref[...] = ce
    sl_ref[...] = sl

    pos_m = t == 1
    neg_m = t == 0
    total_pc = jnp.sum(jnp.where(pos_m, ce, zero))
    total_pl = jnp.sum(jnp.where(pos_m, sl, zero))
    total_nc = jnp.sum(jnp.where(neg_m, ce, zero))
    keyp_ref[...] = jnp.where(pos_m, ce + sl, -1.0)
    keyn_ref[...] = jnp.where(neg_m, ce, -1.0)

    def ext(ref, row, lmask):
        return jnp.sum(jnp.where(lmask, ref[pl.ds(row, 1), :], zero))

    def comp(ka, la, kb, lb):
        take_a = (ka > kb) | ((ka == kb) & (la < lb))
        return jnp.where(take_a, ka, kb), jnp.where(take_a, la, lb)

    def argmax_all(key):
        ka, la = comp(key[0:80], lin[0:80], key[80:160], lin[80:160])
        ka, la = comp(ka[0:40], la[0:40], ka[40:80], la[40:80])
        kb, lb = comp(ka[0:16], la[0:16], ka[16:32], la[16:32])
        kb, lb = comp(kb[0:8], lb[0:8], kb[8:16], lb[8:16])
        kb, lb = comp(kb, lb, ka[32:40], la[32:40])
        for sh in (4, 2, 1):
            kb, lb = comp(kb, lb, jnp.roll(kb, sh, 0), jnp.roll(lb, sh, 0))
        for sh in (64, 32, 16, 8, 4, 2, 1):
            kb, lb = comp(kb, lb, jnp.roll(kb, sh, 1), jnp.roll(lb, sh, 1))
        return kb[0, 0], lb[0, 0]

    def step(key_ref, with_sl, cnt, acc_c, acc_s):
        key = key_ref[...]
        m, i = argmax_all(key)
        has = m >= zero
        ic = jnp.minimum(i, jnp.int32(_RP - 1))
        row = jax.lax.shift_right_logical(ic, 7)
        lmask = lane_iota == jnp.bitwise_and(ic, 127)
        x1i = ext(ax1_ref, row, lmask)
        y1i = ext(ay1_ref, row, lmask)
        x2i = ext(ax2_ref, row, lmask)
        y2i = ext(ay2_ref, row, lmask)
        ce_i = ext(ce_ref, row, lmask)
        area_i = (x2i - x1i) * (y2i - y1i)
        inter = (jnp.maximum(jnp.minimum(x2i, ax2_ref[...])
                             - jnp.maximum(x1i, ax1_ref[...]), zero)
                 * jnp.maximum(jnp.minimum(y2i, ay2_ref[...])
                               - jnp.maximum(y1i, ay1_ref[...]), zero))
        iou = inter / ((area_i + areas_ref[...]) - inter)
        kill = (lin == i) | jnp.logical_not(iou <= _IOU_T)
        key_ref[...] = jnp.where(kill & has, -1.0, key)
        take = has & (cnt < _HALF_BATCH)
        acc_c = acc_c + jnp.where(take, ce_i, zero)
        if with_sl:
            acc_s = acc_s + jnp.where(take, ext(sl_ref, row, lmask), zero)
        cnt = cnt + jnp.where(has, 1, 0)
        return cnt, acc_c, acc_s

    def body(_, st):
        cnt_p, acc_pc, acc_pl, cnt_n, acc_nc = st
        cnt_p, acc_pc, acc_pl = step(keyp_ref, True, cnt_p, acc_pc, acc_pl)
        cnt_n, acc_nc, _ = step(keyn_ref, False, cnt_n, acc_nc, zero)
        return cnt_p, acc_pc, acc_pl, cnt_n, acc_nc

    cnt_p, acc_pc, acc_pl, cnt_n, acc_nc = jax.lax.fori_loop(
        0, _HALF_BATCH + 1, body,
        (jnp.int32(0), zero, zero, jnp.int32(0), zero))

    trunc_p = cnt_p > _HALF_BATCH
    trunc_n = cnt_n > _HALF_BATCH
    keep_p = jnp.minimum(cnt_p, _HALF_BATCH)
    keep_n = jnp.minimum(cnt_n, _HALF_BATCH)
    sum_pc = jnp.where(trunc_p, acc_pc, total_pc)
    sum_pl = jnp.where(trunc_p, acc_pl, total_pl)
    sum_nc = jnp.where(trunc_n, acc_nc, total_nc)
    cls_out[0, 0] = (sum_nc + sum_pc) / (keep_p + keep_n).astype(jnp.float32)
    loc_out[0, 0] = sum_pl / keep_p.astype(jnp.float32)


def kernel(cls_pred, cls_target, loc_pred, loc_target, anchors):
    cp = cls_pred[0]
    ct = cls_target[0, 0].astype(jnp.int32)
    lp = loc_pred[0]
    lt = loc_target[0]
    an = anchors[0]
    pad = _RP - _R

    def p2(x, v=0):
        return jnp.pad(x, (0, pad), constant_values=v).reshape(_ROWS, _LANES)

    args = (p2(cp[:, 0]), p2(cp[:, 1]), p2(ct, 2),
            p2(lp[:, 0]), p2(lp[:, 1]), p2(lt[:, 0]), p2(lt[:, 1]),
            p2(an[:, 0]), p2(an[:, 1]), p2(an[:, 2]), p2(an[:, 3]))
    cls_o, loc_o = pl.pallas_call(
        _ohem_kernel,
        out_shape=(jax.ShapeDtypeStruct((1, 1), jnp.float32),
                   jax.ShapeDtypeStruct((1, 1), jnp.float32)),
        out_specs=(pl.BlockSpec(memory_space=pltpu.SMEM),
                   pl.BlockSpec(memory_space=pltpu.SMEM)),
        scratch_shapes=[pltpu.VMEM((_ROWS, _LANES), jnp.float32)
                        for _ in range(5)],
    )(*args)
    return cls_o[0, 0], loc_o[0, 0]


# speedup vs baseline: 1.2417x; 1.2417x over previous
import jax
import jax.numpy as jnp
from jax.experimental import pallas as pl
from jax.experimental.pallas import tpu as pltpu

_R = 20000
_ROWS = 160
_LANES = 128
_RP = _ROWS * _LANES
_IOU_T = 0.7
_HALF_BATCH = 200
_SIGMA = 10.0


def _ohem_kernel(cls0_ref, cls1_ref, ct_ref, lp0_ref, lp1_ref, lt0_ref,
                 lt1_ref, ax1_ref, ay1_ref, ax2_ref, ay2_ref,
                 cls_out, loc_out,
                 ce_ref, sl_ref, areas_ref, keyp_ref, keyn_ref):
    shape = (_ROWS, _LANES)
    lin = (jax.lax.broadcasted_iota(jnp.int32, shape, 0) * _LANES
           + jax.lax.broadcasted_iota(jnp.int32, shape, 1))
    lane_iota = jax.lax.broadcasted_iota(jnp.int32, (1, _LANES), 1)
    zero = jnp.float32(0.0)
    t = ct_ref[...]

    c0 = cls0_ref[...]
    c1 = cls1_ref[...]
    mx = jnp.maximum(c0, c1)
    s0 = c0 - mx
    s1 = c1 - mx
    lse = jnp.log(jnp.exp(s0) + jnp.exp(s1))
    ce = lse - jnp.where(t == 1, s1, s0)

    def _sl1(d):
        less_one = (d < 1.0 / _SIGMA).astype(jnp.float32)
        return (less_one * 0.5 * d ** 2 * _SIGMA
                + jnp.abs(1 - less_one) * (d - 0.5 / _SIGMA))

    sl = (_sl1(jnp.abs(lt0_ref[...] - lp0_ref[...]))
          + _sl1(jnp.abs(lt1_ref[...] - lp1_ref[...])))

    areas_ref[...] = ((ax2_ref[...] - ax1_ref[...])
                      * (ay2_ref[...] - ay1_ref[...]))
    ce_ref[...] = ce
    sl_ref[...] = sl

    pos_m = t == 1
    neg_m = t == 0
    total_pc = jnp.sum(jnp.where(pos_m, ce, zero))
    total_pl = jnp.sum(jnp.where(pos_m, sl, zero))
    total_nc = jnp.sum(jnp.where(neg_m, ce, zero))
    keyp_ref[...] = jnp.where(pos_m, ce + sl, -1.0)
    keyn_ref[...] = jnp.where(neg_m, ce, -1.0)

    def tourney(key, payload):
        def merge(a, b):
            ta = (a[0] > b[0]) | ((a[0] == b[0]) & (a[1] < b[1]))
            return tuple(jnp.where(ta, x, y) for x, y in zip(a, b))

        def sl(t, i, j):
            return tuple(x[i:j] for x in t)

        t = (key, lin) + payload
        t = merge(sl(t, 0, 80), sl(t, 80, 160))
        t = merge(sl(t, 0, 40), sl(t, 40, 80))
        u = merge(sl(t, 0, 16), sl(t, 16, 32))
        u = merge(sl(u, 0, 8), sl(u, 8, 16))
        u = merge(u, sl(t, 32, 40))
        for sh in (4, 2, 1):
            u = merge(u, tuple(jnp.roll(x, sh, 0) for x in u))
        for sh in (64, 32, 16, 8, 4, 2, 1):
            u = merge(u, tuple(jnp.roll(x, sh, 1) for x in u))
        return tuple(x[0:1] for x in u)

    def body(_, st):
        cnt_p, acc_pc, acc_pl, cnt_n, acc_nc = st
        key_p = keyp_ref[...]
        key_n = keyn_ref[...]
        x1 = ax1_ref[...]
        y1 = ay1_ref[...]
        x2 = ax2_ref[...]
        y2 = ay2_ref[...]
        ce_v = ce_ref[...]
        areas = areas_ref[...]
        kp, ip, x1p, y1p, x2p, y2p, cep, slp = tourney(
            key_p, (x1, y1, x2, y2, ce_v, sl_ref[...]))
        kn, nn, x1n, y1n, x2n, y2n, cen = tourney(
            key_n, (x1, y1, x2, y2, ce_v))
        has_p = kp >= zero
        has_n = kn >= zero
        area_p = (x2p - x1p) * (y2p - y1p)
        area_n = (x2n - x1n) * (y2n - y1n)
        inter_p = (jnp.maximum(jnp.minimum(x2p, x2) - jnp.maximum(x1p, x1), zero)
                   * jnp.maximum(jnp.minimum(y2p, y2) - jnp.maximum(y1p, y1), zero))
        inter_n = (jnp.maximum(jnp.minimum(x2n, x2) - jnp.maximum(x1n, x1), zero)
                   * jnp.maximum(jnp.minimum(y2n, y2) - jnp.maximum(y1n, y1), zero))
        iou_p = inter_p / ((area_p + areas) - inter_p)
        iou_n = inter_n / ((area_n + areas) - inter_n)
        kill_p = (lin == ip) | jnp.logical_not(iou_p <= _IOU_T)
        kill_n = (lin == nn) | jnp.logical_not(iou_n <= _IOU_T)
        keyp_ref[...] = jnp.where(kill_p & has_p, -1.0, key_p)
        keyn_ref[...] = jnp.where(kill_n & has_n, -1.0, key_n)
        take_p = has_p & (cnt_p < _HALF_BATCH)
        take_n = has_n & (cnt_n < _HALF_BATCH)
        acc_pc = acc_pc + jnp.where(take_p, cep, zero)
        acc_pl = acc_pl + jnp.where(take_p, slp, zero)
        acc_nc = acc_nc + jnp.where(take_n, cen, zero)
        cnt_p = cnt_p + jnp.where(has_p, 1, 0)
        cnt_n = cnt_n + jnp.where(has_n, 1, 0)
        return cnt_p, acc_pc, acc_pl, cnt_n, acc_nc

    zrow_f = jnp.zeros((1, _LANES), jnp.float32)
    zrow_i = jnp.zeros((1, _LANES), jnp.int32)
    cnt_pv, acc_pcv, acc_plv, cnt_nv, acc_ncv = jax.lax.fori_loop(
        0, _HALF_BATCH + 1, body,
        (zrow_i, zrow_f, zrow_f, zrow_i, zrow_f))
    cnt_p = cnt_pv[0, 0]
    acc_pc = acc_pcv[0, 0]
    acc_pl = acc_plv[0, 0]
    cnt_n = cnt_nv[0, 0]
    acc_nc = acc_ncv[0, 0]

    trunc_p = cnt_p > _HALF_BATCH
    trunc_n = cnt_n > _HALF_BATCH
    keep_p = jnp.minimum(cnt_p, _HALF_BATCH)
    keep_n = jnp.minimum(cnt_n, _HALF_BATCH)
    sum_pc = jnp.where(trunc_p, acc_pc, total_pc)
    sum_pl = jnp.where(trunc_p, acc_pl, total_pl)
    sum_nc = jnp.where(trunc_n, acc_nc, total_nc)
    cls_out[0, 0] = (sum_nc + sum_pc) / (keep_p + keep_n).astype(jnp.float32)
    loc_out[0, 0] = sum_pl / keep_p.astype(jnp.float32)


def kernel(cls_pred, cls_target, loc_pred, loc_target, anchors):
    cp = cls_pred[0]
    ct = cls_target[0, 0].astype(jnp.int32)
    lp = loc_pred[0]
    lt = loc_target[0]
    an = anchors[0]
    pad = _RP - _R

    def p2(x, v=0):
        return jnp.pad(x, (0, pad), constant_values=v).reshape(_ROWS, _LANES)

    args = (p2(cp[:, 0]), p2(cp[:, 1]), p2(ct, 2),
            p2(lp[:, 0]), p2(lp[:, 1]), p2(lt[:, 0]), p2(lt[:, 1]),
            p2(an[:, 0]), p2(an[:, 1]), p2(an[:, 2]), p2(an[:, 3]))
    cls_o, loc_o = pl.pallas_call(
        _ohem_kernel,
        out_shape=(jax.ShapeDtypeStruct((1, 1), jnp.float32),
                   jax.ShapeDtypeStruct((1, 1), jnp.float32)),
        out_specs=(pl.BlockSpec(memory_space=pltpu.SMEM),
                   pl.BlockSpec(memory_space=pltpu.SMEM)),
        scratch_shapes=[pltpu.VMEM((_ROWS, _LANES), jnp.float32)
                        for _ in range(5)],
    )(*args)
    return cls_o[0, 0], loc_o[0, 0]


# speedup vs baseline: 1.5078x; 1.2143x over previous
import jax
import jax.numpy as jnp
from jax.experimental import pallas as pl
from jax.experimental.pallas import tpu as pltpu

_R = 20000
_ROWS = 160
_LANES = 128
_RP = _ROWS * _LANES
_IOU_T = 0.7
_HALF_BATCH = 200
_SIGMA = 10.0


def _ohem_kernel(cls0_ref, cls1_ref, ct_ref, lp0_ref, lp1_ref, lt0_ref,
                 lt1_ref, ax1_ref, ay1_ref, ax2_ref, ay2_ref,
                 cls_out, loc_out,
                 ce_ref, sl_ref, areas_ref, keyp_ref, keyn_ref):
    shape = (_ROWS, _LANES)
    lin = (jax.lax.broadcasted_iota(jnp.int32, shape, 0) * _LANES
           + jax.lax.broadcasted_iota(jnp.int32, shape, 1))
    lane_iota = jax.lax.broadcasted_iota(jnp.int32, (1, _LANES), 1)
    zero = jnp.float32(0.0)
    t = ct_ref[...]

    c0 = cls0_ref[...]
    c1 = cls1_ref[...]
    mx = jnp.maximum(c0, c1)
    s0 = c0 - mx
    s1 = c1 - mx
    lse = jnp.log(jnp.exp(s0) + jnp.exp(s1))
    ce = lse - jnp.where(t == 1, s1, s0)

    def _sl1(d):
        less_one = (d < 1.0 / _SIGMA).astype(jnp.float32)
        return (less_one * 0.5 * d ** 2 * _SIGMA
                + jnp.abs(1 - less_one) * (d - 0.5 / _SIGMA))

    sl = (_sl1(jnp.abs(lt0_ref[...] - lp0_ref[...]))
          + _sl1(jnp.abs(lt1_ref[...] - lp1_ref[...])))

    areas_ref[...] = ((ax2_ref[...] - ax1_ref[...])
                      * (ay2_ref[...] - ay1_ref[...]))
    ce_ref[...] = ce
    sl_ref[...] = sl

    pos_m = t == 1
    neg_m = t == 0
    total_pc = jnp.sum(jnp.where(pos_m, ce, zero))
    total_pl = jnp.sum(jnp.where(pos_m, sl, zero))
    total_nc = jnp.sum(jnp.where(neg_m, ce, zero))
    keyp_ref[...] = jnp.where(pos_m, ce + sl, -1.0)
    keyn_ref[...] = jnp.where(neg_m, ce, -1.0)

    def ext(ref, row, lmask):
        return jnp.sum(jnp.where(lmask, ref[pl.ds(row, 1), :], zero))

    def gt(ka, la, kb, lb):
        return (ka > kb) | ((ka == kb) & (la < lb))

    def tourney2(key):
        def merge(a, b):
            a1k, a1l, a2k, a2l = a
            b1k, b1l, b2k, b2l = b
            ta = gt(a1k, a1l, b1k, b1l)
            fk = jnp.where(ta, a1k, b1k)
            fl = jnp.where(ta, a1l, b1l)
            lk = jnp.where(ta, b1k, a1k)
            ll = jnp.where(ta, b1l, a1l)
            ck = jnp.where(ta, a2k, b2k)
            cl = jnp.where(ta, a2l, b2l)
            tb = gt(lk, ll, ck, cl)
            return fk, fl, jnp.where(tb, lk, ck), jnp.where(tb, ll, cl)

        def sl4(t, i, j):
            return tuple(x[i:j] for x in t)

        ta = gt(key[0:80], lin[0:80], key[80:160], lin[80:160])
        t = (jnp.where(ta, key[0:80], key[80:160]),
             jnp.where(ta, lin[0:80], lin[80:160]),
             jnp.where(ta, key[80:160], key[0:80]),
             jnp.where(ta, lin[80:160], lin[0:80]))
        t = merge(sl4(t, 0, 40), sl4(t, 40, 80))
        u = merge(sl4(t, 0, 16), sl4(t, 16, 32))
        u = merge(sl4(u, 0, 8), sl4(u, 8, 16))
        u = merge(u, sl4(t, 32, 40))
        for sh in (4, 2, 1):
            u = merge(u, tuple(jnp.roll(x, sh, 0) for x in u))
        for sh in (64, 32, 16, 8, 4, 2, 1):
            u = merge(u, tuple(jnp.roll(x, sh, 1) for x in u))
        return u

    def side(key_ref, with_sl, cnt, acc_c, acc_s):
        key = key_ref[...]
        k1v, l1v, k2v, l2v = tourney2(key)
        k1 = k1v[0, 0]
        i1 = l1v[0, 0]
        k2 = k2v[0, 0]
        i2 = l2v[0, 0]
        has1 = k1 >= zero
        has2 = k2 >= zero

        def extract(i):
            ic = jnp.minimum(i, jnp.int32(_RP - 1))
            row = jax.lax.shift_right_logical(ic, 7)
            lmask = lane_iota == jnp.bitwise_and(ic, 127)
            return (ext(ax1_ref, row, lmask), ext(ay1_ref, row, lmask),
                    ext(ax2_ref, row, lmask), ext(ay2_ref, row, lmask),
                    ext(ce_ref, row, lmask),
                    ext(sl_ref, row, lmask) if with_sl else zero)

        x11, y11, x21, y21, ce1, sl1 = extract(i1)
        x12, y12, x22, y22, ce2, sl2 = extract(i2)
        a1 = (x21 - x11) * (y21 - y11)
        a2 = (x22 - x12) * (y22 - y12)
        i12 = (jnp.maximum(jnp.minimum(x21, x22) - jnp.maximum(x11, x12), zero)
               * jnp.maximum(jnp.minimum(y21, y22) - jnp.maximum(y11, y12), zero))
        iou12 = i12 / ((a1 + a2) - i12)
        valid2 = has2 & (iou12 <= _IOU_T)

        x1 = ax1_ref[...]
        y1 = ay1_ref[...]
        x2 = ax2_ref[...]
        y2 = ay2_ref[...]
        areas = areas_ref[...]
        inter1 = (jnp.maximum(jnp.minimum(x21, x2) - jnp.maximum(x11, x1), zero)
                  * jnp.maximum(jnp.minimum(y21, y2) - jnp.maximum(y11, y1), zero))
        inter2 = (jnp.maximum(jnp.minimum(x22, x2) - jnp.maximum(x12, x1), zero)
                  * jnp.maximum(jnp.minimum(y22, y2) - jnp.maximum(y12, y1), zero))
        iou1 = inter1 / ((a1 + areas) - inter1)
        iou2 = inter2 / ((a2 + areas) - inter2)
        kill = ((has1 & ((lin == i1) | jnp.logical_not(iou1 <= _IOU_T)))
                | (valid2 & ((lin == i2) | jnp.logical_not(iou2 <= _IOU_T))))
        key_ref[...] = jnp.where(kill, -1.0, key)
        take1 = has1 & (cnt < _HALF_BATCH)
        take2 = valid2 & (cnt < _HALF_BATCH - 1)
        acc_c = (acc_c + jnp.where(take1, ce1, zero)
                 + jnp.where(take2, ce2, zero))
        if with_sl:
            acc_s = (acc_s + jnp.where(take1, sl1, zero)
                     + jnp.where(take2, sl2, zero))
        cnt = cnt + jnp.where(has1, 1, 0) + jnp.where(valid2, 1, 0)
        return cnt, acc_c, acc_s, has1

    def cond(st):
        cnt_p, _, _, cnt_n, _, act_p, act_n = st
        return ((act_p & (cnt_p <= _HALF_BATCH))
                | (act_n & (cnt_n <= _HALF_BATCH)))

    def body(st):
        cnt_p, acc_pc, acc_pl, cnt_n, acc_nc, act_p, act_n = st
        cnt_p, acc_pc, acc_pl, h1p = side(keyp_ref, True,
                                          cnt_p, acc_pc, acc_pl)
        cnt_n, acc_nc, _, h1n = side(keyn_ref, False, cnt_n, acc_nc, zero)
        return (cnt_p, acc_pc, acc_pl, cnt_n, acc_nc,
                act_p & h1p, act_n & h1n)

    cnt_p, acc_pc, acc_pl, cnt_n, acc_nc, _, _ = jax.lax.while_loop(
        cond, body,
        (jnp.int32(0), zero, zero, jnp.int32(0), zero,
         jnp.bool_(True), jnp.bool_(True)))

    trunc_p = cnt_p > _HALF_BATCH
    trunc_n = cnt_n > _HALF_BATCH
    keep_p = jnp.minimum(cnt_p, _HALF_BATCH)
    keep_n = jnp.minimum(cnt_n, _HALF_BATCH)
    sum_pc = jnp.where(trunc_p, acc_pc, total_pc)
    sum_pl = jnp.where(trunc_p, acc_pl, total_pl)
    sum_nc = jnp.where(trunc_n, acc_nc, total_nc)
    cls_out[0, 0] = (sum_nc + sum_pc) / (keep_p + keep_n).astype(jnp.float32)
    loc_out[0, 0] = sum_pl / keep_p.astype(jnp.float32)


def kernel(cls_pred, cls_target, loc_pred, loc_target, anchors):
    cp = cls_pred[0]
    ct = cls_target[0, 0].astype(jnp.int32)
    lp = loc_pred[0]
    lt = loc_target[0]
    an = anchors[0]
    pad = _RP - _R

    def p2(x, v=0):
        return jnp.pad(x, (0, pad), constant_values=v).reshape(_ROWS, _LANES)

    args = (p2(cp[:, 0]), p2(cp[:, 1]), p2(ct, 2),
            p2(lp[:, 0]), p2(lp[:, 1]), p2(lt[:, 0]), p2(lt[:, 1]),
            p2(an[:, 0]), p2(an[:, 1]), p2(an[:, 2]), p2(an[:, 3]))
    cls_o, loc_o = pl.pallas_call(
        _ohem_kernel,
        out_shape=(jax.ShapeDtypeStruct((1, 1), jnp.float32),
                   jax.ShapeDtypeStruct((1, 1), jnp.float32)),
        out_specs=(pl.BlockSpec(memory_space=pltpu.SMEM),
                   pl.BlockSpec(memory_space=pltpu.SMEM)),
        scratch_shapes=[pltpu.VMEM((_ROWS, _LANES), jnp.float32)
                        for _ in range(5)],
    )(*args)
    return cls_o[0, 0], loc_o[0, 0]


# speedup vs baseline: 2.1302x; 1.4127x over previous
import jax
import jax.numpy as jnp
from jax.experimental import pallas as pl
from jax.experimental.pallas import tpu as pltpu

_R = 20000
_ROWS = 160
_LANES = 128
_RP = _ROWS * _LANES
_IOU_T = 0.7
_HALF_BATCH = 200
_SIGMA = 10.0


def _ohem_kernel(cls0_ref, cls1_ref, ct_ref, lp0_ref, lp1_ref, lt0_ref,
                 lt1_ref, ax1_ref, ay1_ref, ax2_ref, ay2_ref,
                 cls_out, loc_out,
                 ce_ref, sl_ref, areas_ref, keyp_ref, keyn_ref):
    shape = (_ROWS, _LANES)
    lin = (jax.lax.broadcasted_iota(jnp.int32, shape, 0) * _LANES
           + jax.lax.broadcasted_iota(jnp.int32, shape, 1))
    lane_iota = jax.lax.broadcasted_iota(jnp.int32, (1, _LANES), 1)
    zero = jnp.float32(0.0)
    t = ct_ref[...]

    c0 = cls0_ref[...]
    c1 = cls1_ref[...]
    mx = jnp.maximum(c0, c1)
    s0 = c0 - mx
    s1 = c1 - mx
    lse = jnp.log(jnp.exp(s0) + jnp.exp(s1))
    ce = lse - jnp.where(t == 1, s1, s0)

    def _sl1(d):
        less_one = (d < 1.0 / _SIGMA).astype(jnp.float32)
        return (less_one * 0.5 * d ** 2 * _SIGMA
                + jnp.abs(1 - less_one) * (d - 0.5 / _SIGMA))

    sl = (_sl1(jnp.abs(lt0_ref[...] - lp0_ref[...]))
          + _sl1(jnp.abs(lt1_ref[...] - lp1_ref[...])))

    areas_ref[...] = ((ax2_ref[...] - ax1_ref[...])
                      * (ay2_ref[...] - ay1_ref[...]))
    ce_ref[...] = ce
    sl_ref[...] = sl

    pos_m = t == 1
    neg_m = t == 0
    total_pc = jnp.sum(jnp.where(pos_m, ce, zero))
    total_pl = jnp.sum(jnp.where(pos_m, sl, zero))
    total_nc = jnp.sum(jnp.where(neg_m, ce, zero))
    keyp_ref[...] = jnp.where(pos_m, ce + sl, -1.0)
    keyn_ref[...] = jnp.where(neg_m, ce, -1.0)

    def ext(ref, row, lmask):
        return jnp.sum(jnp.where(lmask, ref[pl.ds(row, 1), :], zero))

    def gt(ka, la, kb, lb):
        return (ka > kb) | ((ka == kb) & (la < lb))

    def ce_full(x, y):
        t = gt(x[0], x[1], y[0], y[1])
        return ((jnp.where(t, x[0], y[0]), jnp.where(t, x[1], y[1])),
                (jnp.where(t, y[0], x[0]), jnp.where(t, y[1], x[1])))

    def ce_max(x, y):
        t = gt(x[0], x[1], y[0], y[1])
        return (jnp.where(t, x[0], y[0]), jnp.where(t, x[1], y[1]))

    def merge22(a, b):
        h1, lo1 = ce_full(a[0], b[1])
        h2, lo2 = ce_full(a[1], b[0])
        c1, c2 = ce_full(h1, h2)
        c3, c4 = ce_full(lo1, lo2)
        return [c1, c2, c3, c4]

    def merge44(a, b):
        h1 = ce_max(a[0], b[3])
        h2 = ce_max(a[1], b[2])
        h3 = ce_max(a[2], b[1])
        h4 = ce_max(a[3], b[0])
        p1, p3 = ce_full(h1, h3)
        p2, p4 = ce_full(h2, h4)
        c1, c2 = ce_full(p1, p2)
        c3, c4 = ce_full(p3, p4)
        return [c1, c2, c3, c4]

    def tourney4(key):
        def sll(t, i, j):
            return [(p[0][i:j], p[1][i:j]) for p in t]

        w, lo = ce_full((key[0:80], lin[0:80]), (key[80:160], lin[80:160]))
        s2 = [w, lo]
        s4 = merge22(sll(s2, 0, 40), sll(s2, 40, 80))
        u = merge44(sll(s4, 0, 16), sll(s4, 16, 32))
        u = merge44(sll(u, 0, 8), sll(u, 8, 16))
        u = merge44(u, sll(s4, 32, 40))
        for sh in (4, 2, 1):
            u = merge44(u, [(jnp.roll(p[0], sh, 0), jnp.roll(p[1], sh, 0))
                            for p in u])
        for sh in (64, 32, 16, 8, 4, 2, 1):
            u = merge44(u, [(jnp.roll(p[0], sh, 1), jnp.roll(p[1], sh, 1))
                            for p in u])
        return u

    def side(key_ref, with_sl, cnt, acc_c, acc_s):
        key = key_ref[...]
        u = tourney4(key)
        ks = [p[0][0, 0] for p in u]
        idx = [p[1][0, 0] for p in u]
        has = [k >= zero for k in ks]

        def extract(i):
            ic = jnp.minimum(i, jnp.int32(_RP - 1))
            row = jax.lax.shift_right_logical(ic, 7)
            lmask = lane_iota == jnp.bitwise_and(ic, 127)
            return (ext(ax1_ref, row, lmask), ext(ay1_ref, row, lmask),
                    ext(ax2_ref, row, lmask), ext(ay2_ref, row, lmask),
                    ext(ce_ref, row, lmask),
                    ext(sl_ref, row, lmask) if with_sl else zero)

        boxes = [extract(i) for i in idx]
        areas_s = [(b[2] - b[0]) * (b[3] - b[1]) for b in boxes]

        def iou_pair(j, k):
            bj, bk = boxes[j], boxes[k]
            w = jnp.maximum(jnp.minimum(bj[2], bk[2])
                            - jnp.maximum(bj[0], bk[0]), zero)
            h = jnp.maximum(jnp.minimum(bj[3], bk[3])
                            - jnp.maximum(bj[1], bk[1]), zero)
            it = w * h
            return it / ((areas_s[j] + areas_s[k]) - it)

        s12 = iou_pair(0, 1) <= _IOU_T
        s13 = iou_pair(0, 2) <= _IOU_T
        s14 = iou_pair(0, 3) <= _IOU_T
        s23 = iou_pair(1, 2) <= _IOU_T
        s24 = iou_pair(1, 3) <= _IOU_T
        s34 = iou_pair(2, 3) <= _IOU_T
        sel1 = has[0]
        sel2 = has[1] & s12
        sel3 = has[2] & s13 & (jnp.logical_not(sel2) | s23)
        sel4 = (has[3] & s14 & (jnp.logical_not(sel2) | s24)
                & (jnp.logical_not(sel3) | s34))
        sels = [sel1, sel2, sel3, sel4]

        x1 = ax1_ref[...]
        y1 = ay1_ref[...]
        x2 = ax2_ref[...]
        y2 = ay2_ref[...]
        areas = areas_ref[...]

        def sweep_kill(j):
            bj = boxes[j]
            inter = (jnp.maximum(jnp.minimum(bj[2], x2)
                                 - jnp.maximum(bj[0], x1), zero)
                     * jnp.maximum(jnp.minimum(bj[3], y2)
                                   - jnp.maximum(bj[1], y1), zero))
            iou = inter / ((areas_s[j] + areas) - inter)
            return sels[j] & ((lin == idx[j])
                              | jnp.logical_not(iou <= _IOU_T))

        kill = (sweep_kill(0) | sweep_kill(1)) | (sweep_kill(2)
                                                  | sweep_kill(3))
        key_ref[...] = jnp.where(kill, -1.0, key)

        rank = cnt
        for j in range(4):
            take = sels[j] & (rank < _HALF_BATCH)
            acc_c = acc_c + jnp.where(take, boxes[j][4], zero)
            if with_sl:
                acc_s = acc_s + jnp.where(take, boxes[j][5], zero)
            rank = rank + jnp.where(sels[j], 1, 0)
        cnt = rank
        return cnt, acc_c, acc_s, has[0]

    def cond(st):
        cnt_p, _, _, cnt_n, _, act_p, act_n = st
        return ((act_p & (cnt_p <= _HALF_BATCH))
                | (act_n & (cnt_n <= _HALF_BATCH)))

    def body(st):
        cnt_p, acc_pc, acc_pl, cnt_n, acc_nc, act_p, act_n = st
        cnt_p, acc_pc, acc_pl, h1p = side(keyp_ref, True,
                                          cnt_p, acc_pc, acc_pl)
        cnt_n, acc_nc, _, h1n = side(keyn_ref, False, cnt_n, acc_nc, zero)
        return (cnt_p, acc_pc, acc_pl, cnt_n, acc_nc,
                act_p & h1p, act_n & h1n)

    cnt_p, acc_pc, acc_pl, cnt_n, acc_nc, _, _ = jax.lax.while_loop(
        cond, body,
        (jnp.int32(0), zero, zero, jnp.int32(0), zero,
         jnp.bool_(True), jnp.bool_(True)))

    trunc_p = cnt_p > _HALF_BATCH
    trunc_n = cnt_n > _HALF_BATCH
    keep_p = jnp.minimum(cnt_p, _HALF_BATCH)
    keep_n = jnp.minimum(cnt_n, _HALF_BATCH)
    sum_pc = jnp.where(trunc_p, acc_pc, total_pc)
    sum_pl = jnp.where(trunc_p, acc_pl, total_pl)
    sum_nc = jnp.where(trunc_n, acc_nc, total_nc)
    cls_out[0, 0] = (sum_nc + sum_pc) / (keep_p + keep_n).astype(jnp.float32)
    loc_out[0, 0] = sum_pl / keep_p.astype(jnp.float32)


def kernel(cls_pred, cls_target, loc_pred, loc_target, anchors):
    cp = cls_pred[0]
    ct = cls_target[0, 0].astype(jnp.int32)
    lp = loc_pred[0]
    lt = loc_target[0]
    an = anchors[0]
    pad = _RP - _R

    def p2(x, v=0):
        return jnp.pad(x, (0, pad), constant_values=v).reshape(_ROWS, _LANES)

    args = (p2(cp[:, 0]), p2(cp[:, 1]), p2(ct, 2),
            p2(lp[:, 0]), p2(lp[:, 1]), p2(lt[:, 0]), p2(lt[:, 1]),
            p2(an[:, 0]), p2(an[:, 1]), p2(an[:, 2]), p2(an[:, 3]))
    cls_o, loc_o = pl.pallas_call(
        _ohem_kernel,
        out_shape=(jax.ShapeDtypeStruct((1, 1), jnp.float32),
                   jax.ShapeDtypeStruct((1, 1), jnp.float32)),
        out_specs=(pl.BlockSpec(memory_space=pltpu.SMEM),
                   pl.BlockSpec(memory_space=pltpu.SMEM)),
        scratch_shapes=[pltpu.VMEM((_ROWS, _LANES), jnp.float32)
                        for _ in range(5)],
    )(*args)
    return cls_o[0, 0], loc_o[0, 0]
